# baseline (speedup 1.0000x reference)
import functools

import jax
import jax.numpy as jnp
from jax import lax
from jax.experimental import pallas as pl
from jax.experimental.pallas import tpu as pltpu
from jax.experimental.pallas import tpu_sc as plsc

N = 10000
E = 320000
H = 128
AVG_D_LOG = 1.0

NT = 32
RNG = 128
NPASS = 3
NP = NT * RNG * NPASS
SCAN = 2000
CAP = SCAN + 127
BF = 64
NCH = E // SCAN
BM1 = 80
EPT = E // NT
FMAX = 3.4028235e38


def _mm_kernel(x_ref, w_ref, b_ref, o_ref, *, act):
    y = jnp.dot(x_ref[...], w_ref[...], preferred_element_type=jnp.float32)
    y = y + b_ref[...]
    if act == "relu":
        y = jnp.maximum(y, 0.0)
    o_ref[...] = y


def _linear(x, W, b, act="none", block=512):
    M, K = x.shape
    F = W.shape[1]
    Mp = (M + block - 1) // block * block
    if Mp != M:
        x = jnp.pad(x, ((0, Mp - M), (0, 0)))
    out = pl.pallas_call(
        functools.partial(_mm_kernel, act=act),
        grid=(Mp // block,),
        in_specs=[
            pl.BlockSpec((block, K), lambda i: (i, 0)),
            pl.BlockSpec((K, F), lambda i: (0, 0)),
            pl.BlockSpec((F,), lambda i: (0,)),
        ],
        out_specs=pl.BlockSpec((block, F), lambda i: (i, 0)),
        out_shape=jax.ShapeDtypeStruct((Mp, F), jnp.float32),
    )(x, W, b)
    return out[:M] if Mp != M else out


def _soft_edge_kernel(m_ref, v3_ref, b3_ref, wse_ref, bse_ref, o_ref):
    m = jnp.dot(m_ref[...], v3_ref[...], preferred_element_type=jnp.float32)
    m = m + b3_ref[...]
    g = jax.nn.sigmoid(jnp.dot(m, wse_ref[...],
                               preferred_element_type=jnp.float32) + bse_ref[...])
    o_ref[...] = m * g


def _soft_edge(m1, V3, b3, Wse, bse, block=1024):
    M = m1.shape[0]
    return pl.pallas_call(
        _soft_edge_kernel,
        grid=(M // block,),
        in_specs=[
            pl.BlockSpec((block, H), lambda i: (i, 0)),
            pl.BlockSpec((H, H), lambda i: (0, 0)),
            pl.BlockSpec((H,), lambda i: (0,)),
            pl.BlockSpec((H, 1), lambda i: (0, 0)),
            pl.BlockSpec((1,), lambda i: (0,)),
        ],
        out_specs=pl.BlockSpec((block, H), lambda i: (i, 0)),
        out_shape=jax.ShapeDtypeStruct((M, H), jnp.float32),
    )(m1, V3, b3, Wse, bse)


def _post_kernel(h_ref, sB, qB, xB, nB, dB, sC, qC, xC, nC, dC,
                 w_ref, b_ref, o_ref):
    h = h_ref[...]
    W = w_ref[...]
    y = jnp.dot(h, W[:H], preferred_element_type=jnp.float32) + b_ref[...] + h

    def accum(y, s_, q_, x_, n_, d_, o0):
        deg = jnp.sum(d_[...], axis=1, keepdims=True)
        degc = jnp.maximum(deg, 1.0)
        mask = deg > 0.0
        mean = jnp.where(mask, s_[...] / degc, 0.0)
        msq = q_[...] / degc
        std = jnp.sqrt(jnp.maximum(msq - (s_[...] / degc) ** 2, 0.0) + 1e-5)
        std = jnp.where(mask, std, 0.0)
        mx = jnp.where(mask, x_[...], 0.0)
        mn = jnp.where(mask, n_[...], 0.0)
        logd = jnp.log(deg + 1.0)
        amp_s = logd / AVG_D_LOG
        att_s = AVG_D_LOG / jnp.where(mask, logd, 1.0)
        for j, arr in enumerate((mean, mx, mn, std)):
            y = y + jnp.dot(arr, W[o0 + j * H: o0 + (j + 1) * H],
                            preferred_element_type=jnp.float32)
            y = y + jnp.dot(arr * amp_s, W[o0 + (4 + j) * H: o0 + (5 + j) * H],
                            preferred_element_type=jnp.float32)
            y = y + jnp.dot(arr * att_s, W[o0 + (8 + j) * H: o0 + (9 + j) * H],
                            preferred_element_type=jnp.float32)
        return y

    y = accum(y, sB, qB, xB, nB, dB, H)
    y = accum(y, sC, qC, xC, nC, dC, 13 * H)
    o_ref[...] = y


def _post(h, sB, qB, xB, nB, dB, sC, qC, xC, nC, dC, Wp, bp, block=1536):
    spec = lambda w: pl.BlockSpec((block, w), lambda i: (i, 0))
    return pl.pallas_call(
        _post_kernel,
        grid=(NP // block,),
        in_specs=[spec(H), spec(H), spec(H), spec(H), spec(H), spec(16),
                  spec(H), spec(H), spec(H), spec(H), spec(16),
                  pl.BlockSpec((25 * H, H), lambda i: (0, 0)),
                  pl.BlockSpec((H,), lambda i: (0,))],
        out_specs=spec(H),
        out_shape=jax.ShapeDtypeStruct((NP, H), jnp.float32),
    )(h, sB, qB, xB, nB, dB, sC, qC, xC, nC, dC, Wp, bp)


def _readout_kernel(h_ref, w1, b1, w2, b2, o_ref):
    hh = h_ref[...]
    sm = jnp.sum(hh, axis=0, keepdims=True)
    mx = jnp.max(hh, axis=0, keepdims=True)
    g = jnp.concatenate([sm, sm / N, mx], axis=-1)
    g = jnp.maximum(jnp.dot(g, w1[...], preferred_element_type=jnp.float32)
                    + b1[...], 0.0)
    o_ref[...] = jnp.dot(g, w2[...], preferred_element_type=jnp.float32) + b2[...]


def _readout(h2, W1, b1, W2, b2):
    return pl.pallas_call(
        _readout_kernel,
        grid=(1,),
        in_specs=[
            pl.BlockSpec((N, H), lambda i: (0, 0)),
            pl.BlockSpec((3 * H, H), lambda i: (0, 0)),
            pl.BlockSpec((H,), lambda i: (0,)),
            pl.BlockSpec((H, H), lambda i: (0, 0)),
            pl.BlockSpec((H,), lambda i: (0,)),
        ],
        out_specs=pl.BlockSpec((1, H), lambda i: (0, 0)),
        out_shape=jax.ShapeDtypeStruct((1, H), jnp.float32),
    )(h2, W1, b1, W2, b2)


def _mesh():
    return plsc.VectorSubcoreMesh(core_axis_name="c", subcore_axis_name="s")


def _vsplat(vec, idx):
    return lax.gather(
        vec, idx[:, None],
        dimension_numbers=lax.GatherDimensionNumbers(
            offset_dims=(), collapsed_slice_dims=(0,), start_index_map=(0,)),
        slice_sizes=(1,), mode=lax.GatherScatterMode.PROMISE_IN_BOUNDS)


def _prefix16(x):
    iota = lax.iota(jnp.int32, 16)
    s = x
    for k in (1, 2, 4, 8):
        shifted = _vsplat(s, jnp.maximum(iota - k, 0))
        s = s + jnp.where(iota >= k, shifted, 0)
    return s


def _agg_body(nsrc, src_h, dst_h, tbl_h, d_h, sum_o, sq_o, mx_o, mn_o,
              dst_c0, src_c0, dst_c1, src_c1, st_eid, st_src, st_dstl,
              ixA0, ixB0, ixD0, ixA1, ixB1, ixD1,
              bufA0, bufB0, bufD0, bufA1, bufB1, bufD1,
              acc_sum, acc_sq, acc_mx, acc_mn,
              semD0, semD1, semG0, semG1):
    c = lax.axis_index("c")
    s = lax.axis_index("s")
    wid = s * 2 + c
    iota = lax.iota(jnp.int32, 16)
    ones_i = jnp.ones((16,), jnp.int32)
    idx15 = jnp.full((16,), 15, jnp.int32)
    zero16 = jnp.zeros((16,), jnp.float32)
    neg16 = jnp.full((16,), -FMAX, jnp.float32)
    pos16 = jnp.full((16,), FMAX, jnp.float32)
    NPAIR = NCH // 2
    sets = ((ixA0, ixB0, ixD0, bufA0, bufB0, bufD0, semG0),
            (ixA1, ixB1, ixD1, bufA1, bufB1, bufD1, semG1))

    def issue(ch, dc, sc, sem):
        base = ch * SCAN
        pltpu.async_copy(dst_h.at[pl.ds(base, SCAN)], dc, sem)
        if nsrc:
            pltpu.async_copy(src_h.at[pl.ds(base, SCAN)], sc, sem)

    def drain(dc, sc, sem):
        pltpu.make_async_copy(dst_h.at[pl.ds(0, SCAN)], dc, sem).wait()
        if nsrc:
            pltpu.make_async_copy(src_h.at[pl.ds(0, SCAN)], sc, sem).wait()

    def pass_body(p, pcarry):
        lo = (p * NT + wid) * RNG
        lo_v = jnp.full((16,), lo, jnp.int32)
        hi_v = lo_v + RNG

        def init_row(r, carry):
            for k in range(8):
                sl = pl.ds(k * 16, 16)
                acc_sum[r, sl] = zero16
                acc_sq[r, sl] = zero16
                acc_mx[r, sl] = neg16
                acc_mn[r, sl] = pos16
            return carry
        lax.fori_loop(0, RNG, init_row, 0)

        def stage_issue(off, fill_v, masked, si):
            ixA, ixB, ixD, bufA, bufB, bufD, semG = sets[si]
            for v in range(BF // 16):
                slv = pl.ds(v * 16, 16)
                offv = pl.ds(off + v * 16, 16)
                ev = st_eid[offv]
                dlv = st_dstl[offv]
                sv = st_src[offv] if nsrc else None
                if masked:
                    vmv = (jnp.full((16,), off + v * 16, jnp.int32)
                           + iota) < fill_v
                    ev = jnp.where(vmv, ev, 0)
                    dlv = jnp.where(vmv, dlv, 0)
                    if nsrc:
                        sv = jnp.where(vmv, sv, 0)
                ixD[slv] = ev
                if nsrc:
                    ixA[slv] = sv * 4
                    ixB[slv] = (dlv + lo_v) * 4 + 1
            if nsrc:
                pltpu.async_copy(tbl_h.at[ixA], bufA, semG)
                pltpu.async_copy(tbl_h.at[ixB], bufB, semG)
                pltpu.async_copy(d_h.at[ixD], bufD, semG)
            else:
                pltpu.async_copy(d_h.at[ixD], bufA, semG)

        def gdrain(si):
            ixA, ixB, ixD, bufA, bufB, bufD, semG = sets[si]
            pltpu.make_async_copy(d_h.at[pl.ds(0, BF)], bufA, semG).wait()
            if nsrc:
                pltpu.make_async_copy(d_h.at[pl.ds(0, BF)], bufB,
                                      semG).wait()
                pltpu.make_async_copy(d_h.at[pl.ds(0, BF)], bufD,
                                      semG).wait()

        def rmw(off, fill_v, masked, si):
            ixA, ixB, ixD, bufA, bufB, bufD, semG = sets[si]

            def group(g, carry):
                for e in range(8):
                    pe = g * 8 + e
                    idx_e = jnp.full((16,), off + pe, jnp.int32)
                    row = plsc.load_gather(st_dstl, [idx_e])
                    vm = None
                    if masked:
                        vm = idx_e < fill_v
                        row = jnp.where(vm, row, 0)
                    for k in range(8):
                        col = iota + (16 * k)
                        sl = pl.ds(16 * k, 16)
                        if nsrc:
                            m = bufA[pe, sl] + bufB[pe, sl] + bufD[pe, sl]
                        else:
                            m = bufA[pe, sl]
                        plsc.addupdate_scatter(acc_sum, [row, col], m,
                                               mask=vm)
                        plsc.addupdate_scatter(acc_sq, [row, col], m * m,
                                               mask=vm)
                        cx = plsc.load_gather(acc_mx, [row, col])
                        plsc.store_scatter(acc_mx, [row, col],
                                           jnp.maximum(cx, m), mask=vm)
                        cn = plsc.load_gather(acc_mn, [row, col])
                        plsc.store_scatter(acc_mn, [row, col],
                                           jnp.minimum(cn, m), mask=vm)
                return carry
            lax.fori_loop(0, BF // 8, group, 0)

        def do_chunk(dc, sc, ch, fill_v):
            base_v = jnp.full((16,), ch * SCAN, jnp.int32)

            def scan_block(sb, fill_v):
                for u in range(25):
                    start = sb * 400 + u * 16
                    sl = pl.ds(start, 16)
                    dv = dc[sl]
                    mask = (dv >= lo_v) & (dv < hi_v)
                    csum = _prefix16(jnp.where(mask, ones_i, 0))
                    cnt = _vsplat(csum, idx15)
                    pos = fill_v + csum - 1
                    eidv = base_v + (jnp.full((16,), start, jnp.int32)
                                     + iota)
                    plsc.store_scatter(st_eid, [pos], eidv, mask=mask)
                    if nsrc:
                        plsc.store_scatter(st_src, [pos], sc[sl], mask=mask)
                    plsc.store_scatter(st_dstl, [pos], dv - lo_v, mask=mask)
                    fill_v = fill_v + cnt
                return fill_v
            fill_v = lax.fori_loop(0, SCAN // 400, scan_block, fill_v)
            fill_s = jnp.max(fill_v)
            nb2 = fill_s // (2 * BF)

            @pl.when(nb2 > 0)
            def _():
                stage_issue(0, fill_v, False, 0)

            def fpair(i, carry):
                off0 = i * (2 * BF)
                gdrain(0)
                stage_issue(off0 + BF, fill_v, False, 1)
                rmw(off0, fill_v, False, 0)
                gdrain(1)

                @pl.when(i + 1 < nb2)
                def _():
                    stage_issue(off0 + 2 * BF, fill_v, False, 0)
                rmw(off0 + BF, fill_v, False, 1)
                return carry
            lax.fori_loop(0, nb2, fpair, 0)
            rem = fill_s - nb2 * (2 * BF)
            rem_v = jnp.full((16,), rem, jnp.int32)
            for j in range(8):
                dpos = iota + 16 * j
                gidx = nb2 * (2 * BF) + dpos
                rmask = dpos < rem_v
                ev = plsc.load_gather(st_eid, [gidx], mask=rmask)
                plsc.store_scatter(st_eid, [dpos], ev, mask=rmask)
                if nsrc:
                    sv = plsc.load_gather(st_src, [gidx], mask=rmask)
                    plsc.store_scatter(st_src, [dpos], sv, mask=rmask)
                dlv = plsc.load_gather(st_dstl, [gidx], mask=rmask)
                plsc.store_scatter(st_dstl, [dpos], dlv, mask=rmask)
            return jnp.full((16,), rem, jnp.int32)

        issue(0, dst_c0, src_c0, semD0)

        def pair_body(cp, fill_v):
            issue(2 * cp + 1, dst_c1, src_c1, semD1)
            drain(dst_c0, src_c0, semD0)
            fill_v = do_chunk(dst_c0, src_c0, 2 * cp, fill_v)

            @pl.when(cp + 1 < NPAIR)
            def _():
                issue(2 * cp + 2, dst_c0, src_c0, semD0)
            drain(dst_c1, src_c1, semD1)
            fill_v = do_chunk(dst_c1, src_c1, 2 * cp + 1, fill_v)
            return fill_v

        fill_v = lax.fori_loop(0, NPAIR, pair_body,
                               jnp.zeros((16,), jnp.int32))
        for off0 in (0, BF):
            stage_issue(off0, fill_v, True, 0)
            gdrain(0)
            rmw(off0, fill_v, True, 0)

        pltpu.sync_copy(acc_sum, sum_o.at[pl.ds(lo, RNG)])
        pltpu.sync_copy(acc_sq, sq_o.at[pl.ds(lo, RNG)])
        pltpu.sync_copy(acc_mx, mx_o.at[pl.ds(lo, RNG)])
        pltpu.sync_copy(acc_mn, mn_o.at[pl.ds(lo, RNG)])
        return pcarry

    lax.fori_loop(0, NPASS, pass_body, 0)


def _make_agg(nsrc):
    return functools.partial(
        pl.kernel,
        functools.partial(_agg_body, nsrc),
        out_type=[jax.ShapeDtypeStruct((NP, H), jnp.float32)] * 4,
        mesh=_mesh(),
        compiler_params=pltpu.CompilerParams(needs_layout_passes=False),
        scratch_types=(
            [pltpu.VMEM((SCAN,), jnp.int32)] * 4
            + [pltpu.VMEM((CAP,), jnp.int32)] * 3
            + [pltpu.VMEM((BF,), jnp.int32)] * 6
            + [pltpu.VMEM((BF, H), jnp.float32)] * 6
            + [pltpu.VMEM((RNG, H), jnp.float32)] * 4
            + [pltpu.SemaphoreType.DMA] * 4
        ),
    )()


def _deg_body(dst_h, deg_o, dst_c, acc):
    c = lax.axis_index("c")
    s = lax.axis_index("s")
    wid = s * 2 + c
    iota = lax.iota(jnp.int32, 16)
    los = [(p * NT + wid) * RNG for p in range(NPASS)]
    lo_vs = [jnp.full((16,), lo, jnp.int32) for lo in los]
    ones_f = jnp.ones((16,), jnp.float32)
    zero16 = jnp.zeros((16,), jnp.float32)

    def init_row(r, carry):
        acc[r, pl.ds(0, 16)] = zero16
        return carry
    lax.fori_loop(0, NPASS * RNG, init_row, 0)

    def chunk_body(ch, carry):
        pltpu.sync_copy(dst_h.at[pl.ds(ch * SCAN, SCAN)], dst_c)
        for v in range(SCAN // 16):
            dv = dst_c[pl.ds(v * 16, 16)]
            for p in range(NPASS):
                mp = (dv >= lo_vs[p]) & (dv < lo_vs[p] + RNG)
                plsc.addupdate_scatter(acc, [(dv - lo_vs[p]) + p * RNG, iota],
                                       ones_f, mask=mp)
        return carry
    lax.fori_loop(0, NCH, chunk_body, 0)
    for p in range(NPASS):
        pltpu.sync_copy(acc.at[pl.ds(p * RNG, RNG)],
                        deg_o.at[pl.ds(los[p], RNG)])


def _deg(dst):
    return pl.kernel(
        _deg_body,
        out_type=jax.ShapeDtypeStruct((NP, 16), jnp.float32),
        mesh=_mesh(),
        compiler_params=pltpu.CompilerParams(needs_layout_passes=False),
        scratch_types=[
            pltpu.VMEM((SCAN,), jnp.int32),
            pltpu.VMEM((NPASS * RNG, 16), jnp.float32),
        ],
    )(dst)


def _m1_body(src_h, dst_h, tbl_h, b1_h, m1_o,
             src_c, dst_c, idxP, idxQ, bufP, bufQ, stg, bvm, sem):
    c = lax.axis_index("c")
    s = lax.axis_index("s")
    wid = s * 2 + c
    ebase = wid * EPT
    pltpu.sync_copy(b1_h, bvm)

    def batch(bi, carry):
        base = ebase + bi * BM1
        pltpu.sync_copy(src_h.at[pl.ds(base, BM1)], src_c)
        pltpu.sync_copy(dst_h.at[pl.ds(base, BM1)], dst_c)
        for v in range(BM1 // 16):
            sl = pl.ds(v * 16, 16)
            idxP[sl] = src_c[sl] * 4 + 2
            idxQ[sl] = dst_c[sl] * 4 + 3
        cp = pltpu.async_copy(tbl_h.at[idxP], bufP, sem)
        cq = pltpu.async_copy(tbl_h.at[idxQ], bufQ, sem)
        cp.wait()
        cq.wait()
        for e in range(BM1):
            for k in range(8):
                sl = pl.ds(k * 16, 16)
                stg[e, sl] = jnp.maximum(bufP[e, sl] + bufQ[e, sl] + bvm[sl],
                                         0.0)
        pltpu.sync_copy(stg, m1_o.at[pl.ds(base, BM1)])
        return carry
    lax.fori_loop(0, EPT // BM1, batch, 0)


def _m1(srcc, dstc, tbl4, c1):
    return pl.kernel(
        _m1_body,
        out_type=jax.ShapeDtypeStruct((E, H), jnp.float32),
        mesh=_mesh(),
        compiler_params=pltpu.CompilerParams(needs_layout_passes=False),
        scratch_types=[
            pltpu.VMEM((BM1,), jnp.int32),
            pltpu.VMEM((BM1,), jnp.int32),
            pltpu.VMEM((BM1,), jnp.int32),
            pltpu.VMEM((BM1,), jnp.int32),
            pltpu.VMEM((BM1, H), jnp.float32),
            pltpu.VMEM((BM1, H), jnp.float32),
            pltpu.VMEM((BM1, H), jnp.float32),
            pltpu.VMEM((H,), jnp.float32),
            pltpu.SemaphoreType.DMA,
        ],
    )(srcc, dstc, tbl4, c1)


def kernel(x, edge_attr, params, edge_index_bond, edge_index_complete):
    src_b, dst_b = edge_index_bond[0], edge_index_bond[1]
    srcc, dstc = edge_index_complete[0], edge_index_complete[1]
    (Wn, bn), = params['node_in']
    (We, be), = params['edge_in']
    h = _linear(x, Wn, bn, act="relu")
    eh = _linear(edge_attr, We, be, act="relu")
    h = jnp.pad(h, ((0, NP - N), (0, 0)))
    degB = _deg(dst_b)
    degC = _deg(dstc)
    for p in params['layers']:
        (Wp1, bp1), = p['pretrans']
        (V1, c1), (V2, c2) = p['pretrans_complete']
        Wse, bse = p['soft_edge']
        (Wpost, bpost), = p['posttrans']
        Wcat = jnp.concatenate([Wp1[:H], Wp1[H:2 * H], V1[:H], V1[H:]], axis=1)
        tbl = _linear(h, Wcat, jnp.zeros((4 * H,), jnp.float32))
        tbl4 = tbl.reshape(NP * 4, H)
        D = _linear(eh, Wp1[2 * H:], bp1)
        sB, qB, xB, nB = _make_agg(True)(src_b, dst_b, tbl4, D)
        m1 = _m1(srcc, dstc, tbl4, c1)
        ec = _soft_edge(m1, V2, c2, Wse, bse)
        sC, qC, xC, nC = _make_agg(False)(dstc, dstc, tbl4, ec)
        h = _post(h, sB, qB, xB, nB, degB, sC, qC, xC, nC, degC, Wpost, bpost)
    (Wo1, bo1), (Wo2, bo2) = params['node_out']
    h2 = _linear(_linear(h[:N], Wo1, bo1, act="relu"), Wo2, bo2)
    (Wr1, br1), (Wr2, br2) = params['readout']
    return _readout(h2, Wr1, br1, Wr2, br2).reshape(H)


# speedup vs baseline: 1.2349x; 1.2349x over previous
import functools

import jax
import jax.numpy as jnp
from jax import lax
from jax.experimental import pallas as pl
from jax.experimental.pallas import tpu as pltpu
from jax.experimental.pallas import tpu_sc as plsc

N = 10000
E = 320000
H = 128
AVG_D_LOG = 1.0

NT = 32
RNG = 160
NPASS = 2
NP = NT * RNG * NPASS
SCAN = 3200
CAP = SCAN + 63
BF = 64
NCH = E // SCAN
BM1 = 80
EPT = E // NT
FMAX = 3.4028235e38


def _mm_kernel(x_ref, w_ref, b_ref, o_ref, *, act):
    y = jnp.dot(x_ref[...], w_ref[...], preferred_element_type=jnp.float32)
    y = y + b_ref[...]
    if act == "relu":
        y = jnp.maximum(y, 0.0)
    o_ref[...] = y


def _linear(x, W, b, act="none", block=512):
    M, K = x.shape
    F = W.shape[1]
    Mp = (M + block - 1) // block * block
    if Mp != M:
        x = jnp.pad(x, ((0, Mp - M), (0, 0)))
    out = pl.pallas_call(
        functools.partial(_mm_kernel, act=act),
        grid=(Mp // block,),
        in_specs=[
            pl.BlockSpec((block, K), lambda i: (i, 0)),
            pl.BlockSpec((K, F), lambda i: (0, 0)),
            pl.BlockSpec((F,), lambda i: (0,)),
        ],
        out_specs=pl.BlockSpec((block, F), lambda i: (i, 0)),
        out_shape=jax.ShapeDtypeStruct((Mp, F), jnp.float32),
    )(x, W, b)
    return out[:M] if Mp != M else out


def _soft_edge_kernel(m_ref, v3_ref, b3_ref, wse_ref, bse_ref, o_ref):
    m = jnp.dot(m_ref[...], v3_ref[...], preferred_element_type=jnp.float32)
    m = m + b3_ref[...]
    g = jax.nn.sigmoid(jnp.dot(m, wse_ref[...],
                               preferred_element_type=jnp.float32) + bse_ref[...])
    o_ref[...] = m * g


def _soft_edge(m1, V3, b3, Wse, bse, block=1024):
    M = m1.shape[0]
    return pl.pallas_call(
        _soft_edge_kernel,
        grid=(M // block,),
        in_specs=[
            pl.BlockSpec((block, H), lambda i: (i, 0)),
            pl.BlockSpec((H, H), lambda i: (0, 0)),
            pl.BlockSpec((H,), lambda i: (0,)),
            pl.BlockSpec((H, 1), lambda i: (0, 0)),
            pl.BlockSpec((1,), lambda i: (0,)),
        ],
        out_specs=pl.BlockSpec((block, H), lambda i: (i, 0)),
        out_shape=jax.ShapeDtypeStruct((M, H), jnp.float32),
    )(m1, V3, b3, Wse, bse)


def _post_kernel(h_ref, sB, qB, xB, nB, dB, sC, qC, xC, nC, dC,
                 w_ref, b_ref, o_ref):
    h = h_ref[...]
    W = w_ref[...]
    y = jnp.dot(h, W[:H], preferred_element_type=jnp.float32) + b_ref[...] + h

    def accum(y, s_, q_, x_, n_, d_, o0):
        deg = jnp.sum(d_[...], axis=1, keepdims=True)
        degc = jnp.maximum(deg, 1.0)
        mask = deg > 0.0
        mean = jnp.where(mask, s_[...] / degc, 0.0)
        msq = q_[...] / degc
        std = jnp.sqrt(jnp.maximum(msq - (s_[...] / degc) ** 2, 0.0) + 1e-5)
        std = jnp.where(mask, std, 0.0)
        mx = jnp.where(mask, x_[...], 0.0)
        mn = jnp.where(mask, n_[...], 0.0)
        logd = jnp.log(deg + 1.0)
        amp_s = logd / AVG_D_LOG
        att_s = AVG_D_LOG / jnp.where(mask, logd, 1.0)
        for j, arr in enumerate((mean, mx, mn, std)):
            y = y + jnp.dot(arr, W[o0 + j * H: o0 + (j + 1) * H],
                            preferred_element_type=jnp.float32)
            y = y + jnp.dot(arr * amp_s, W[o0 + (4 + j) * H: o0 + (5 + j) * H],
                            preferred_element_type=jnp.float32)
            y = y + jnp.dot(arr * att_s, W[o0 + (8 + j) * H: o0 + (9 + j) * H],
                            preferred_element_type=jnp.float32)
        return y

    y = accum(y, sB, qB, xB, nB, dB, H)
    y = accum(y, sC, qC, xC, nC, dC, 13 * H)
    o_ref[...] = y


def _post(h, sB, qB, xB, nB, dB, sC, qC, xC, nC, dC, Wp, bp, block=2560):
    spec = lambda w: pl.BlockSpec((block, w), lambda i: (i, 0))
    return pl.pallas_call(
        _post_kernel,
        grid=(NP // block,),
        in_specs=[spec(H), spec(H), spec(H), spec(H), spec(H), spec(16),
                  spec(H), spec(H), spec(H), spec(H), spec(16),
                  pl.BlockSpec((25 * H, H), lambda i: (0, 0)),
                  pl.BlockSpec((H,), lambda i: (0,))],
        out_specs=spec(H),
        out_shape=jax.ShapeDtypeStruct((NP, H), jnp.float32),
    )(h, sB, qB, xB, nB, dB, sC, qC, xC, nC, dC, Wp, bp)


def _readout_kernel(h_ref, w1, b1, w2, b2, o_ref):
    hh = h_ref[...]
    sm = jnp.sum(hh, axis=0, keepdims=True)
    mx = jnp.max(hh, axis=0, keepdims=True)
    g = jnp.concatenate([sm, sm / N, mx], axis=-1)
    g = jnp.maximum(jnp.dot(g, w1[...], preferred_element_type=jnp.float32)
                    + b1[...], 0.0)
    o_ref[...] = jnp.dot(g, w2[...], preferred_element_type=jnp.float32) + b2[...]


def _readout(h2, W1, b1, W2, b2):
    return pl.pallas_call(
        _readout_kernel,
        grid=(1,),
        in_specs=[
            pl.BlockSpec((N, H), lambda i: (0, 0)),
            pl.BlockSpec((3 * H, H), lambda i: (0, 0)),
            pl.BlockSpec((H,), lambda i: (0,)),
            pl.BlockSpec((H, H), lambda i: (0, 0)),
            pl.BlockSpec((H,), lambda i: (0,)),
        ],
        out_specs=pl.BlockSpec((1, H), lambda i: (0, 0)),
        out_shape=jax.ShapeDtypeStruct((1, H), jnp.float32),
    )(h2, W1, b1, W2, b2)


def _mesh():
    return plsc.VectorSubcoreMesh(core_axis_name="c", subcore_axis_name="s")


def _vsplat(vec, idx):
    return lax.gather(
        vec, idx[:, None],
        dimension_numbers=lax.GatherDimensionNumbers(
            offset_dims=(), collapsed_slice_dims=(0,), start_index_map=(0,)),
        slice_sizes=(1,), mode=lax.GatherScatterMode.PROMISE_IN_BOUNDS)


def _prefix16(x):
    iota = lax.iota(jnp.int32, 16)
    s = x
    for k in (1, 2, 4, 8):
        shifted = _vsplat(s, jnp.maximum(iota - k, 0))
        s = s + jnp.where(iota >= k, shifted, 0)
    return s


def _agg_body(nsrc, src_h, dst_h, tbl_h, d_h, sum_o, sq_o, mx_o, mn_o,
              dst_c0, src_c0, dst_c1, src_c1, st_eid, st_src, st_dstl,
              ixA, ixB, ixD, bufA, bufB, bufD,
              acc_sum, acc_sq, acc_mx, acc_mn, semD0, semD1, semG):
    c = lax.axis_index("c")
    s = lax.axis_index("s")
    wid = s * 2 + c
    iota = lax.iota(jnp.int32, 16)
    ones_i = jnp.ones((16,), jnp.int32)
    idx15 = jnp.full((16,), 15, jnp.int32)
    zero16 = jnp.zeros((16,), jnp.float32)
    neg16 = jnp.full((16,), -FMAX, jnp.float32)
    pos16 = jnp.full((16,), FMAX, jnp.float32)
    NPAIR = NCH // 2

    def issue(ch, dc, sc, sem):
        base = ch * SCAN
        pltpu.async_copy(dst_h.at[pl.ds(base, SCAN)], dc, sem)
        if nsrc:
            pltpu.async_copy(src_h.at[pl.ds(base, SCAN)], sc, sem)

    def drain(dc, sc, sem):
        pltpu.make_async_copy(dst_h.at[pl.ds(0, SCAN)], dc, sem).wait()
        if nsrc:
            pltpu.make_async_copy(src_h.at[pl.ds(0, SCAN)], sc, sem).wait()

    def pass_body(p, pcarry):
        lo = (p * NT + wid) * RNG
        lo_v = jnp.full((16,), lo, jnp.int32)
        hi_v = lo_v + RNG

        def init_row(r, carry):
            for k in range(8):
                sl = pl.ds(k * 16, 16)
                acc_sum[r, sl] = zero16
                acc_sq[r, sl] = zero16
                acc_mx[r, sl] = neg16
                acc_mn[r, sl] = pos16
            return carry
        lax.fori_loop(0, RNG, init_row, 0)

        def flush(off, fill_v, masked):
            for v in range(BF // 16):
                slv = pl.ds(v * 16, 16)
                offv = pl.ds(off + v * 16, 16)
                ev = st_eid[offv]
                dlv = st_dstl[offv]
                sv = st_src[offv] if nsrc else None
                if masked:
                    vmv = (jnp.full((16,), off + v * 16, jnp.int32)
                           + iota) < fill_v
                    ev = jnp.where(vmv, ev, 0)
                    dlv = jnp.where(vmv, dlv, 0)
                    if nsrc:
                        sv = jnp.where(vmv, sv, 0)
                ixD[slv] = ev
                if nsrc:
                    ixA[slv] = sv * 4
                    ixB[slv] = (dlv + lo_v) * 4 + 1
            if nsrc:
                ca = pltpu.async_copy(tbl_h.at[ixA], bufA, semG)
                cb = pltpu.async_copy(tbl_h.at[ixB], bufB, semG)
                cd = pltpu.async_copy(d_h.at[ixD], bufD, semG)
                ca.wait()
                cb.wait()
                cd.wait()
            else:
                pltpu.async_copy(d_h.at[ixD], bufA, semG).wait()

            def group(g, carry):
                for e in range(8):
                    pe = g * 8 + e
                    idx_e = jnp.full((16,), off + pe, jnp.int32)
                    row = plsc.load_gather(st_dstl, [idx_e])
                    vm = None
                    if masked:
                        vm = idx_e < fill_v
                        row = jnp.where(vm, row, 0)
                    for k in range(8):
                        col = iota + (16 * k)
                        sl = pl.ds(16 * k, 16)
                        if nsrc:
                            m = bufA[pe, sl] + bufB[pe, sl] + bufD[pe, sl]
                        else:
                            m = bufA[pe, sl]
                        plsc.addupdate_scatter(acc_sum, [row, col], m,
                                               mask=vm)
                        plsc.addupdate_scatter(acc_sq, [row, col], m * m,
                                               mask=vm)
                        cx = plsc.load_gather(acc_mx, [row, col])
                        plsc.store_scatter(acc_mx, [row, col],
                                           jnp.maximum(cx, m), mask=vm)
                        cn = plsc.load_gather(acc_mn, [row, col])
                        plsc.store_scatter(acc_mn, [row, col],
                                           jnp.minimum(cn, m), mask=vm)
                return carry
            lax.fori_loop(0, BF // 8, group, 0)

        def do_chunk(dc, sc, ch, fill_v):
            base_v = jnp.full((16,), ch * SCAN, jnp.int32)

            def scan_block(sb, fill_v):
                for u in range(25):
                    start = sb * 400 + u * 16
                    sl = pl.ds(start, 16)
                    dv = dc[sl]
                    mask = (dv >= lo_v) & (dv < hi_v)
                    csum = _prefix16(jnp.where(mask, ones_i, 0))
                    cnt = _vsplat(csum, idx15)
                    pos = fill_v + csum - 1
                    eidv = base_v + (jnp.full((16,), start, jnp.int32)
                                     + iota)
                    plsc.store_scatter(st_eid, [pos], eidv, mask=mask)
                    if nsrc:
                        plsc.store_scatter(st_src, [pos], sc[sl], mask=mask)
                    plsc.store_scatter(st_dstl, [pos], dv - lo_v, mask=mask)
                    fill_v = fill_v + cnt
                return fill_v
            fill_v = lax.fori_loop(0, SCAN // 400, scan_block, fill_v)
            fill_s = jnp.max(fill_v)
            nb = fill_s // BF

            def fb(i, carry):
                flush(i * BF, fill_v, False)
                return carry
            lax.fori_loop(0, nb, fb, 0)
            rem = fill_s - nb * BF
            rem_v = jnp.full((16,), rem, jnp.int32)
            for j in range(4):
                dpos = iota + 16 * j
                gidx = nb * BF + dpos
                rmask = dpos < rem_v
                ev = plsc.load_gather(st_eid, [gidx], mask=rmask)
                plsc.store_scatter(st_eid, [dpos], ev, mask=rmask)
                if nsrc:
                    sv = plsc.load_gather(st_src, [gidx], mask=rmask)
                    plsc.store_scatter(st_src, [dpos], sv, mask=rmask)
                dlv = plsc.load_gather(st_dstl, [gidx], mask=rmask)
                plsc.store_scatter(st_dstl, [dpos], dlv, mask=rmask)
            return jnp.full((16,), rem, jnp.int32)

        issue(0, dst_c0, src_c0, semD0)

        def pair_body(cp, fill_v):
            issue(2 * cp + 1, dst_c1, src_c1, semD1)
            drain(dst_c0, src_c0, semD0)
            fill_v = do_chunk(dst_c0, src_c0, 2 * cp, fill_v)

            @pl.when(cp + 1 < NPAIR)
            def _():
                issue(2 * cp + 2, dst_c0, src_c0, semD0)
            drain(dst_c1, src_c1, semD1)
            fill_v = do_chunk(dst_c1, src_c1, 2 * cp + 1, fill_v)
            return fill_v

        fill_v = lax.fori_loop(0, NPAIR, pair_body,
                               jnp.zeros((16,), jnp.int32))
        flush(0, fill_v, True)

        pltpu.sync_copy(acc_sum, sum_o.at[pl.ds(lo, RNG)])
        pltpu.sync_copy(acc_sq, sq_o.at[pl.ds(lo, RNG)])
        pltpu.sync_copy(acc_mx, mx_o.at[pl.ds(lo, RNG)])
        pltpu.sync_copy(acc_mn, mn_o.at[pl.ds(lo, RNG)])
        return pcarry

    lax.fori_loop(0, NPASS, pass_body, 0)


def _make_agg(nsrc):
    return functools.partial(
        pl.kernel,
        functools.partial(_agg_body, nsrc),
        out_type=[jax.ShapeDtypeStruct((NP, H), jnp.float32)] * 4,
        mesh=_mesh(),
        compiler_params=pltpu.CompilerParams(needs_layout_passes=False),
        scratch_types=(
            [pltpu.VMEM((SCAN,), jnp.int32)] * 4
            + [pltpu.VMEM((CAP,), jnp.int32)] * 3
            + [pltpu.VMEM((BF,), jnp.int32)] * 3
            + [pltpu.VMEM((BF, H), jnp.float32)] * 3
            + [pltpu.VMEM((RNG, H), jnp.float32)] * 4
            + [pltpu.SemaphoreType.DMA] * 3
        ),
    )()


def _deg_body(dst_h, deg_o, dst_c, acc):
    c = lax.axis_index("c")
    s = lax.axis_index("s")
    wid = s * 2 + c
    iota = lax.iota(jnp.int32, 16)
    los = [(p * NT + wid) * RNG for p in range(NPASS)]
    lo_vs = [jnp.full((16,), lo, jnp.int32) for lo in los]
    ones_f = jnp.ones((16,), jnp.float32)
    zero16 = jnp.zeros((16,), jnp.float32)

    def init_row(r, carry):
        acc[r, pl.ds(0, 16)] = zero16
        return carry
    lax.fori_loop(0, NPASS * RNG, init_row, 0)

    def chunk_body(ch, carry):
        pltpu.sync_copy(dst_h.at[pl.ds(ch * SCAN, SCAN)], dst_c)
        for v in range(SCAN // 16):
            dv = dst_c[pl.ds(v * 16, 16)]
            for p in range(NPASS):
                mp = (dv >= lo_vs[p]) & (dv < lo_vs[p] + RNG)
                plsc.addupdate_scatter(acc, [(dv - lo_vs[p]) + p * RNG, iota],
                                       ones_f, mask=mp)
        return carry
    lax.fori_loop(0, NCH, chunk_body, 0)
    for p in range(NPASS):
        pltpu.sync_copy(acc.at[pl.ds(p * RNG, RNG)],
                        deg_o.at[pl.ds(los[p], RNG)])


def _deg(dst):
    return pl.kernel(
        _deg_body,
        out_type=jax.ShapeDtypeStruct((NP, 16), jnp.float32),
        mesh=_mesh(),
        compiler_params=pltpu.CompilerParams(needs_layout_passes=False),
        scratch_types=[
            pltpu.VMEM((SCAN,), jnp.int32),
            pltpu.VMEM((NPASS * RNG, 16), jnp.float32),
        ],
    )(dst)


def _m1_body(src_h, dst_h, tbl_h, b1_h, m1_o,
             src_c, dst_c, idxP, idxQ, bufP, bufQ, stg, bvm, sem):
    c = lax.axis_index("c")
    s = lax.axis_index("s")
    wid = s * 2 + c
    ebase = wid * EPT
    pltpu.sync_copy(b1_h, bvm)

    def batch(bi, carry):
        base = ebase + bi * BM1
        pltpu.sync_copy(src_h.at[pl.ds(base, BM1)], src_c)
        pltpu.sync_copy(dst_h.at[pl.ds(base, BM1)], dst_c)
        for v in range(BM1 // 16):
            sl = pl.ds(v * 16, 16)
            idxP[sl] = src_c[sl] * 4 + 2
            idxQ[sl] = dst_c[sl] * 4 + 3
        cp = pltpu.async_copy(tbl_h.at[idxP], bufP, sem)
        cq = pltpu.async_copy(tbl_h.at[idxQ], bufQ, sem)
        cp.wait()
        cq.wait()
        for e in range(BM1):
            for k in range(8):
                sl = pl.ds(k * 16, 16)
                stg[e, sl] = jnp.maximum(bufP[e, sl] + bufQ[e, sl] + bvm[sl],
                                         0.0)
        pltpu.sync_copy(stg, m1_o.at[pl.ds(base, BM1)])
        return carry
    lax.fori_loop(0, EPT // BM1, batch, 0)


def _m1(srcc, dstc, tbl4, c1):
    return pl.kernel(
        _m1_body,
        out_type=jax.ShapeDtypeStruct((E, H), jnp.float32),
        mesh=_mesh(),
        compiler_params=pltpu.CompilerParams(needs_layout_passes=False),
        scratch_types=[
            pltpu.VMEM((BM1,), jnp.int32),
            pltpu.VMEM((BM1,), jnp.int32),
            pltpu.VMEM((BM1,), jnp.int32),
            pltpu.VMEM((BM1,), jnp.int32),
            pltpu.VMEM((BM1, H), jnp.float32),
            pltpu.VMEM((BM1, H), jnp.float32),
            pltpu.VMEM((BM1, H), jnp.float32),
            pltpu.VMEM((H,), jnp.float32),
            pltpu.SemaphoreType.DMA,
        ],
    )(srcc, dstc, tbl4, c1)


def kernel(x, edge_attr, params, edge_index_bond, edge_index_complete):
    src_b, dst_b = edge_index_bond[0], edge_index_bond[1]
    srcc, dstc = edge_index_complete[0], edge_index_complete[1]
    (Wn, bn), = params['node_in']
    (We, be), = params['edge_in']
    h = _linear(x, Wn, bn, act="relu")
    eh = _linear(edge_attr, We, be, act="relu")
    h = jnp.pad(h, ((0, NP - N), (0, 0)))
    degB = _deg(dst_b)
    degC = _deg(dstc)
    for p in params['layers']:
        (Wp1, bp1), = p['pretrans']
        (V1, c1), (V2, c2) = p['pretrans_complete']
        Wse, bse = p['soft_edge']
        (Wpost, bpost), = p['posttrans']
        Wcat = jnp.concatenate([Wp1[:H], Wp1[H:2 * H], V1[:H], V1[H:]], axis=1)
        tbl = _linear(h, Wcat, jnp.zeros((4 * H,), jnp.float32))
        tbl4 = tbl.reshape(NP * 4, H)
        D = _linear(eh, Wp1[2 * H:], bp1)
        sB, qB, xB, nB = _make_agg(True)(src_b, dst_b, tbl4, D)
        m1 = _m1(srcc, dstc, tbl4, c1)
        ec = _soft_edge(m1, V2, c2, Wse, bse)
        sC, qC, xC, nC = _make_agg(False)(dstc, dstc, tbl4, ec)
        h = _post(h, sB, qB, xB, nB, degB, sC, qC, xC, nC, degC, Wpost, bpost)
    (Wo1, bo1), (Wo2, bo2) = params['node_out']
    h2 = _linear(_linear(h[:N], Wo1, bo1, act="relu"), Wo2, bo2)
    (Wr1, br1), (Wr2, br2) = params['readout']
    return _readout(h2, Wr1, br1, Wr2, br2).reshape(H)
